# baseline (speedup 1.0000x reference)
import functools
import math

import jax
import jax.numpy as jnp
from jax import lax
from jax.experimental import pallas as pl
from jax.experimental.pallas import tpu as pltpu
from jax.experimental.pallas import tpu_sc as plsc

_PAD = 0
_SMOOTHING = 0.1
_CONF = 1.0 - _SMOOTHING

_L = 128
_W = 2048
_NS = _W // _L

_SC_CORES = 2
_SC_SUBCORES = 16
_NW = _SC_CORES * _SC_SUBCORES

_SC_BLOCKS = 16
_SCCW = 2048

_NBUF = 4
_NSPLIT = 2


def _sc_kernel(x, t32, sc_c0, sc_c1):
    b, _ = x.shape
    rpw = b // _NW
    nch = (sc_c1 - sc_c0) // _SCCW
    mesh = plsc.VectorSubcoreMesh(core_axis_name="c", subcore_axis_name="s")

    @functools.partial(
        pl.kernel,
        mesh=mesh,
        out_type=[
            jax.ShapeDtypeStruct((b,), jnp.float32),
            jax.ShapeDtypeStruct((b,), jnp.float32),
            jax.ShapeDtypeStruct((b, 16), jnp.float32),
        ],
        scratch_types=[
            pltpu.VMEM((rpw,), jnp.int32),
            pltpu.VMEM((rpw * 8, 128), jnp.float32),
            pltpu.VMEM((rpw, 128), jnp.float32),
            pltpu.VMEM((rpw,), jnp.float32),
            pltpu.VMEM((rpw,), jnp.float32),
            pltpu.VMEM((8, _SCCW), jnp.float32),
            pltpu.VMEM((8, _SCCW), jnp.float32),
            pltpu.VMEM((rpw, 16), jnp.float32),
            pltpu.SemaphoreType.DMA,
            pltpu.SemaphoreType.DMA,
        ],
        compiler_params=pltpu.CompilerParams(needs_layout_passes=False),
    )
    def sc_kernel(x_hbm, t_hbm, g_hbm, x0_hbm, srow_hbm,
                  tbuf, tiles, x0chunk, gout, x0out, dbuf0, dbuf1, srow_acc,
                  sem, sem2):
        wid = lax.axis_index("s") * _SC_CORES + lax.axis_index("c")
        base = wid * rpw
        pltpu.sync_copy(t_hbm.at[pl.ds(base, rpw)], tbuf)
        x0cp = pltpu.async_copy(
            x_hbm.at[pl.ds(base, rpw), pl.ds(0, 128)], x0chunk, sem
        )
        copies = []
        for h in range(rpw // 16):
            startv = (tbuf[pl.ds(h * 16, 16)] >> 7) << 7
            for jj in range(16):
                j = h * 16 + jj
                copies.append(pltpu.async_copy(
                    x_hbm.at[pl.ds(base + (j // 8) * 8, 8),
                             pl.ds(pl.multiple_of(startv[jj], 128), 128)],
                    tiles.at[pl.ds(j * 8, 8)],
                    sem,
                ))
        x0cp.wait()
        for cp in copies:
            cp.wait()
        iota = lax.iota(jnp.int32, 16)
        for h in range(rpw // 16):
            jvec = h * 16 + iota
            tvec = tbuf[pl.ds(h * 16, 16)]
            rowidx = jvec * 8 + lax.bitwise_and(base + jvec, 7)
            lanes = lax.bitwise_and(tvec, 127)
            gout[pl.ds(h * 16, 16)] = plsc.load_gather(tiles, [rowidx, lanes])
            x0out[pl.ds(h * 16, 16)] = plsc.load_gather(x0chunk, [jvec, iota * 0])
        pltpu.sync_copy(gout, g_hbm.at[pl.ds(base, rpw)])
        pltpu.sync_copy(x0out, x0_hbm.at[pl.ds(base, rpw)])

        zero16 = jnp.zeros((16,), jnp.float32)
        dbufs = (dbuf0, dbuf1)

        def _acc_chunk(buf, accs):
            def bodyf(i, acc):
                off = pl.multiple_of(i * 16, 16)
                return tuple(a + buf[r, pl.ds(off, 16)]
                             for r, a in enumerate(acc))
            return pl.loop(0, _SCCW // 16, init_carry=accs, unroll=2)(bodyf)

        for a in range(rpw // 8):
            rows0 = base + a * 8
            cps = [None, None]
            cps[0] = pltpu.async_copy(
                x_hbm.at[pl.ds(rows0, 8), pl.ds(sc_c0, _SCCW)], dbuf0, sem2)
            accs = (zero16,) * 8
            for ch in range(nch):
                if ch + 1 < nch:
                    cps[(ch + 1) % 2] = pltpu.async_copy(
                        x_hbm.at[pl.ds(rows0, 8),
                                 pl.ds(sc_c0 + (ch + 1) * _SCCW, _SCCW)],
                        dbufs[(ch + 1) % 2], sem2)
                cps[ch % 2].wait()
                accs = _acc_chunk(dbufs[ch % 2], accs)
            for r in range(8):
                srow_acc[a * 8 + r, :] = accs[r]
        pltpu.sync_copy(srow_acc, srow_hbm.at[pl.ds(base, rpw)])

    return sc_kernel(x, t32)


def _make_tc_body(b, size, n_full, tail_block, smooth, c1):
    rs = b // _NSPLIT

    def _tree_sum(xv, ns):
        vals = [xv[:, s * _L:(s + 1) * _L] for s in range(ns)]
        while len(vals) > 1:
            if len(vals) % 2:
                vals[-2] = vals[-2] + vals[-1]
                vals = vals[:-1]
            vals = [a + b_ for a, b_ in zip(vals[::2], vals[1::2])]
        return vals[0]

    def _body(tail_ref, x_hbm, acc_ref, *rest):
        bufs, sems = rest[:_NBUF], rest[_NBUF]
        c = pl.program_id(0)

        def _copy(nb, s, k):
            col = pl.multiple_of(nb * _W, _W)
            return pltpu.make_async_copy(
                x_hbm.at[pl.ds(k * rs, rs), pl.ds(col, _W)],
                bufs[s].at[pl.ds(k * rs, rs)],
                sems.at[s, k],
            )

        @pl.when(c == 0)
        def _():
            for bidx in range(min(_NBUF, n_full)):
                for k in range(_NSPLIT):
                    _copy(bidx, bidx, k).start()

        for s in range(_NBUF):

            @pl.when(lax.rem(c, _NBUF) == s)
            def _(s=s):
                for k in range(_NSPLIT):
                    _copy(c, s, k).wait()
                tsum = _tree_sum(bufs[s][...], _NS)

                @pl.when(c == 0)
                def _():
                    acc_ref[...] = tsum

                @pl.when(c > 0)
                def _():
                    acc_ref[...] += tsum

                nb = c + _NBUF

                @pl.when(nb < n_full)
                def _():
                    for k in range(_NSPLIT):
                        _copy(nb, s, k).start()

        @pl.when(c == n_full - 1)
        def _():
            cols = tail_block * _W + lax.broadcasted_iota(jnp.int32, (1, _W), 1)
            xv = jnp.where(cols < size, tail_ref[...], 0.0)
            acc_ref[...] += _tree_sum(xv, _NS)

    return _body


def _tc_partial(x, n_full, tail_block):
    b, size = x.shape
    return pl.pallas_call(
        _make_tc_body(b, size, n_full, tail_block, None, None),
        grid=(n_full,),
        in_specs=[
            pl.BlockSpec((b, _W), lambda c: (0, tail_block)),
            pl.BlockSpec(memory_space=pl.ANY),
        ],
        out_specs=pl.BlockSpec((b, _L), lambda c: (0, 0)),
        out_shape=jax.ShapeDtypeStruct((b, _L), jnp.float32),
        scratch_shapes=[pltpu.VMEM((b, _W), jnp.float32) for _ in range(_NBUF)]
        + [pltpu.SemaphoreType.DMA((_NBUF, _NSPLIT))],
        compiler_params=pltpu.CompilerParams(
            dimension_semantics=("arbitrary",),
        ),
    )(x, x)


def _make_combine_body(smooth, c1):
    def _body(t_ref, g_ref, x0_ref, ssc_ref, stc_ref, out_ref):
        t = t_ref[...]
        g = g_ref[...]
        srow = (jnp.sum(stc_ref[...], axis=1, keepdims=True)
                + jnp.sum(ssc_ref[...], axis=1, keepdims=True))
        mask = (t != _PAD).astype(jnp.float32)
        contrib = c1 - smooth * (srow - x0_ref[...] - g) - _CONF * g
        out_ref[0, 0] = jnp.sum(mask * contrib)

    return _body


def kernel(x, target):
    b, size = x.shape
    total_full = size // _W
    n_full = total_full - _SC_BLOCKS
    sc_c0 = n_full * _W
    sc_c1 = total_full * _W
    smooth = _SMOOTHING / (size - 2)
    c1 = (size - 2) * smooth * math.log(smooth) + _CONF * math.log(_CONF)
    t32 = target.astype(jnp.int32)
    g, x0, srow_sc = _sc_kernel(x, t32, sc_c0, sc_c1)
    srow_tc = _tc_partial(x, n_full, total_full)
    out = pl.pallas_call(
        _make_combine_body(smooth, c1),
        in_specs=[
            pl.BlockSpec((b, 1), lambda: (0, 0)),
            pl.BlockSpec((b, 1), lambda: (0, 0)),
            pl.BlockSpec((b, 1), lambda: (0, 0)),
            pl.BlockSpec((b, 16), lambda: (0, 0)),
            pl.BlockSpec((b, _L), lambda: (0, 0)),
        ],
        out_specs=pl.BlockSpec((1, 1), lambda: (0, 0), memory_space=pltpu.SMEM),
        out_shape=jax.ShapeDtypeStruct((1, 1), jnp.float32),
    )(t32.reshape(b, 1), g.reshape(b, 1), x0.reshape(b, 1), srow_sc, srow_tc)
    return out[0, 0]


# speedup vs baseline: 1.0039x; 1.0039x over previous
import functools
import math

import jax
import jax.numpy as jnp
from jax import lax
from jax.experimental import pallas as pl
from jax.experimental.pallas import tpu as pltpu
from jax.experimental.pallas import tpu_sc as plsc

_PAD = 0
_SMOOTHING = 0.1
_CONF = 1.0 - _SMOOTHING

_L = 128
_W = 2048
_NS = _W // _L

_SC_CORES = 2
_SC_SUBCORES = 16
_NW = _SC_CORES * _SC_SUBCORES

_SC_BLOCKS = 16
_SCCW = 2048

_NBUF = 4
_NSPLIT = 2


def _sc_kernel(x, t32, sc_c0, sc_c1):
    b, _ = x.shape
    rpw = b // _NW
    nch = (sc_c1 - sc_c0) // _SCCW
    mesh = plsc.VectorSubcoreMesh(core_axis_name="c", subcore_axis_name="s")

    @functools.partial(
        pl.kernel,
        mesh=mesh,
        out_type=[
            jax.ShapeDtypeStruct((b,), jnp.float32),
            jax.ShapeDtypeStruct((b,), jnp.float32),
            jax.ShapeDtypeStruct((b, 16), jnp.float32),
        ],
        scratch_types=[
            pltpu.VMEM((rpw,), jnp.int32),
            pltpu.VMEM((rpw * 8, 128), jnp.float32),
            pltpu.VMEM((rpw, 128), jnp.float32),
            pltpu.VMEM((rpw,), jnp.float32),
            pltpu.VMEM((rpw,), jnp.float32),
            pltpu.VMEM((8, _SCCW), jnp.float32),
            pltpu.VMEM((8, _SCCW), jnp.float32),
            pltpu.VMEM((rpw, 16), jnp.float32),
            pltpu.SemaphoreType.DMA,
            pltpu.SemaphoreType.DMA,
        ],
        compiler_params=pltpu.CompilerParams(needs_layout_passes=False, skip_device_barrier=True),
    )
    def sc_kernel(x_hbm, t_hbm, g_hbm, x0_hbm, srow_hbm,
                  tbuf, tiles, x0chunk, gout, x0out, dbuf0, dbuf1, srow_acc,
                  sem, sem2):
        wid = lax.axis_index("s") * _SC_CORES + lax.axis_index("c")
        base = wid * rpw
        pltpu.sync_copy(t_hbm.at[pl.ds(base, rpw)], tbuf)
        x0cp = pltpu.async_copy(
            x_hbm.at[pl.ds(base, rpw), pl.ds(0, 128)], x0chunk, sem
        )
        copies = []
        for h in range(rpw // 16):
            startv = (tbuf[pl.ds(h * 16, 16)] >> 7) << 7
            for jj in range(16):
                j = h * 16 + jj
                copies.append(pltpu.async_copy(
                    x_hbm.at[pl.ds(base + (j // 8) * 8, 8),
                             pl.ds(pl.multiple_of(startv[jj], 128), 128)],
                    tiles.at[pl.ds(j * 8, 8)],
                    sem,
                ))
        x0cp.wait()
        for cp in copies:
            cp.wait()
        iota = lax.iota(jnp.int32, 16)
        for h in range(rpw // 16):
            jvec = h * 16 + iota
            tvec = tbuf[pl.ds(h * 16, 16)]
            rowidx = jvec * 8 + lax.bitwise_and(base + jvec, 7)
            lanes = lax.bitwise_and(tvec, 127)
            gout[pl.ds(h * 16, 16)] = plsc.load_gather(tiles, [rowidx, lanes])
            x0out[pl.ds(h * 16, 16)] = plsc.load_gather(x0chunk, [jvec, iota * 0])
        pltpu.sync_copy(gout, g_hbm.at[pl.ds(base, rpw)])
        pltpu.sync_copy(x0out, x0_hbm.at[pl.ds(base, rpw)])

        zero16 = jnp.zeros((16,), jnp.float32)
        dbufs = (dbuf0, dbuf1)

        def _acc_chunk(buf, accs):
            def bodyf(i, acc):
                off = pl.multiple_of(i * 16, 16)
                return tuple(a + buf[r, pl.ds(off, 16)]
                             for r, a in enumerate(acc))
            return pl.loop(0, _SCCW // 16, init_carry=accs, unroll=2)(bodyf)

        for a in range(rpw // 8):
            rows0 = base + a * 8
            cps = [None, None]
            cps[0] = pltpu.async_copy(
                x_hbm.at[pl.ds(rows0, 8), pl.ds(sc_c0, _SCCW)], dbuf0, sem2)
            accs = (zero16,) * 8
            for ch in range(nch):
                if ch + 1 < nch:
                    cps[(ch + 1) % 2] = pltpu.async_copy(
                        x_hbm.at[pl.ds(rows0, 8),
                                 pl.ds(sc_c0 + (ch + 1) * _SCCW, _SCCW)],
                        dbufs[(ch + 1) % 2], sem2)
                cps[ch % 2].wait()
                accs = _acc_chunk(dbufs[ch % 2], accs)
            for r in range(8):
                srow_acc[a * 8 + r, :] = accs[r]
        pltpu.sync_copy(srow_acc, srow_hbm.at[pl.ds(base, rpw)])

    return sc_kernel(x, t32)


def _make_tc_body(b, size, n_full, tail_block, smooth, c1):
    rs = b // _NSPLIT

    def _tree_sum(xv, ns):
        vals = [xv[:, s * _L:(s + 1) * _L] for s in range(ns)]
        while len(vals) > 1:
            if len(vals) % 2:
                vals[-2] = vals[-2] + vals[-1]
                vals = vals[:-1]
            vals = [a + b_ for a, b_ in zip(vals[::2], vals[1::2])]
        return vals[0]

    def _body(tail_ref, x_hbm, acc_ref, *rest):
        bufs, sems = rest[:_NBUF], rest[_NBUF]
        c = pl.program_id(0)

        def _copy(nb, s, k):
            col = pl.multiple_of(nb * _W, _W)
            return pltpu.make_async_copy(
                x_hbm.at[pl.ds(k * rs, rs), pl.ds(col, _W)],
                bufs[s].at[pl.ds(k * rs, rs)],
                sems.at[s, k],
            )

        @pl.when(c == 0)
        def _():
            for bidx in range(min(_NBUF, n_full)):
                for k in range(_NSPLIT):
                    _copy(bidx, bidx, k).start()

        for s in range(_NBUF):

            @pl.when(lax.rem(c, _NBUF) == s)
            def _(s=s):
                for k in range(_NSPLIT):
                    _copy(c, s, k).wait()
                tsum = _tree_sum(bufs[s][...], _NS)

                @pl.when(c == 0)
                def _():
                    acc_ref[...] = tsum

                @pl.when(c > 0)
                def _():
                    acc_ref[...] += tsum

                nb = c + _NBUF

                @pl.when(nb < n_full)
                def _():
                    for k in range(_NSPLIT):
                        _copy(nb, s, k).start()

        @pl.when(c == n_full - 1)
        def _():
            cols = tail_block * _W + lax.broadcasted_iota(jnp.int32, (1, _W), 1)
            xv = jnp.where(cols < size, tail_ref[...], 0.0)
            acc_ref[...] += _tree_sum(xv, _NS)

    return _body


def _tc_partial(x, n_full, tail_block):
    b, size = x.shape
    return pl.pallas_call(
        _make_tc_body(b, size, n_full, tail_block, None, None),
        grid=(n_full,),
        in_specs=[
            pl.BlockSpec((b, _W), lambda c: (0, tail_block)),
            pl.BlockSpec(memory_space=pl.ANY),
        ],
        out_specs=pl.BlockSpec((b, _L), lambda c: (0, 0)),
        out_shape=jax.ShapeDtypeStruct((b, _L), jnp.float32),
        scratch_shapes=[pltpu.VMEM((b, _W), jnp.float32) for _ in range(_NBUF)]
        + [pltpu.SemaphoreType.DMA((_NBUF, _NSPLIT))],
        compiler_params=pltpu.CompilerParams(
            dimension_semantics=("arbitrary",),
            skip_device_barrier=True,
        ),
    )(x, x)


def _make_combine_body(smooth, c1):
    def _body(t_ref, g_ref, x0_ref, ssc_ref, stc_ref, out_ref):
        t = t_ref[...]
        g = g_ref[...]
        srow = (jnp.sum(stc_ref[...], axis=1, keepdims=True)
                + jnp.sum(ssc_ref[...], axis=1, keepdims=True))
        mask = (t != _PAD).astype(jnp.float32)
        contrib = c1 - smooth * (srow - x0_ref[...] - g) - _CONF * g
        out_ref[0, 0] = jnp.sum(mask * contrib)

    return _body


def kernel(x, target):
    b, size = x.shape
    total_full = size // _W
    n_full = total_full - _SC_BLOCKS
    sc_c0 = n_full * _W
    sc_c1 = total_full * _W
    smooth = _SMOOTHING / (size - 2)
    c1 = (size - 2) * smooth * math.log(smooth) + _CONF * math.log(_CONF)
    t32 = target.astype(jnp.int32)
    g, x0, srow_sc = _sc_kernel(x, t32, sc_c0, sc_c1)
    srow_tc = _tc_partial(x, n_full, total_full)
    out = pl.pallas_call(
        _make_combine_body(smooth, c1),
        in_specs=[
            pl.BlockSpec((b, 1), lambda: (0, 0)),
            pl.BlockSpec((b, 1), lambda: (0, 0)),
            pl.BlockSpec((b, 1), lambda: (0, 0)),
            pl.BlockSpec((b, 16), lambda: (0, 0)),
            pl.BlockSpec((b, _L), lambda: (0, 0)),
        ],
        out_specs=pl.BlockSpec((1, 1), lambda: (0, 0), memory_space=pltpu.SMEM),
        out_shape=jax.ShapeDtypeStruct((1, 1), jnp.float32),
    )(t32.reshape(b, 1), g.reshape(b, 1), x0.reshape(b, 1), srow_sc, srow_tc)
    return out[0, 0]


# speedup vs baseline: 1.0098x; 1.0058x over previous
import functools
import math

import jax
import jax.numpy as jnp
from jax import lax
from jax.experimental import pallas as pl
from jax.experimental.pallas import tpu as pltpu
from jax.experimental.pallas import tpu_sc as plsc

_PAD = 0
_SMOOTHING = 0.1
_CONF = 1.0 - _SMOOTHING

_L = 128
_RB = 64

_SC_CORES = 2
_SC_SUBCORES = 16
_NW = _SC_CORES * _SC_SUBCORES


def _sc_gather(x, t32):
    b, _ = x.shape
    rpw = b // _NW
    mesh = plsc.VectorSubcoreMesh(core_axis_name="c", subcore_axis_name="s")

    @functools.partial(
        pl.kernel,
        mesh=mesh,
        out_type=[
            jax.ShapeDtypeStruct((b,), jnp.float32),
            jax.ShapeDtypeStruct((b,), jnp.float32),
        ],
        scratch_types=[
            pltpu.VMEM((rpw,), jnp.int32),
            pltpu.VMEM((rpw * 8, 128), jnp.float32),
            pltpu.VMEM((rpw, 128), jnp.float32),
            pltpu.VMEM((rpw,), jnp.float32),
            pltpu.VMEM((rpw,), jnp.float32),
            pltpu.SemaphoreType.DMA,
        ],
        compiler_params=pltpu.CompilerParams(needs_layout_passes=False),
    )
    def sc_kernel(x_hbm, t_hbm, g_hbm, x0_hbm, tbuf, tiles, x0chunk, gout, x0out, sem):
        wid = lax.axis_index("s") * _SC_CORES + lax.axis_index("c")
        base = wid * rpw
        pltpu.sync_copy(t_hbm.at[pl.ds(base, rpw)], tbuf)
        x0cp = pltpu.async_copy(
            x_hbm.at[pl.ds(base, rpw), pl.ds(0, 128)], x0chunk, sem
        )
        copies = []
        for h in range(rpw // 16):
            startv = (tbuf[pl.ds(h * 16, 16)] >> 7) << 7
            for jj in range(16):
                j = h * 16 + jj
                copies.append(pltpu.async_copy(
                    x_hbm.at[pl.ds(base + (j // 8) * 8, 8),
                             pl.ds(pl.multiple_of(startv[jj], 128), 128)],
                    tiles.at[pl.ds(j * 8, 8)],
                    sem,
                ))
        x0cp.wait()
        for cp in copies:
            cp.wait()
        iota = lax.iota(jnp.int32, 16)
        for h in range(rpw // 16):
            jvec = h * 16 + iota
            tvec = tbuf[pl.ds(h * 16, 16)]
            rowidx = jvec * 8 + lax.bitwise_and(base + jvec, 7)
            lanes = lax.bitwise_and(tvec, 127)
            gout[pl.ds(h * 16, 16)] = plsc.load_gather(tiles, [rowidx, lanes])
            x0out[pl.ds(h * 16, 16)] = plsc.load_gather(x0chunk, [jvec, iota * 0])
        pltpu.sync_copy(gout, g_hbm.at[pl.ds(base, rpw)])
        pltpu.sync_copy(x0out, x0_hbm.at[pl.ds(base, rpw)])

    return sc_kernel(x, t32)


def _make_tc_body(b, size, n_steps, smooth, c1):
    nfull = size // _L
    rem = size - nfull * _L

    def _body(t_ref, g_ref, x0_ref, x_ref, out_ref, srow_ref):
        c = pl.program_id(0)
        xblk = x_ref[...]
        vals = [xblk[:, s * _L:(s + 1) * _L] for s in range(nfull)]
        while len(vals) > 1:
            if len(vals) % 2:
                vals[-2] = vals[-2] + vals[-1]
                vals = vals[:-1]
            vals = [a + b_ for a, b_ in zip(vals[::2], vals[1::2])]
        srow = jnp.sum(vals[0], axis=1, keepdims=True)
        if rem:
            srow = srow + jnp.sum(
                xblk[:, nfull * _L:], axis=1, keepdims=True
            )
        srow_ref[pl.ds(c * _RB, _RB), :] = srow

        @pl.when(c == n_steps - 1)
        def _():
            t = t_ref[...]
            g = g_ref[...]
            s = srow_ref[...]
            mask = (t != _PAD).astype(jnp.float32)
            contrib = c1 - smooth * (s - x0_ref[...] - g) - _CONF * g
            out_ref[0, 0] = jnp.sum(mask * contrib)

    return _body


def kernel(x, target):
    b, size = x.shape
    n_steps = b // _RB
    smooth = _SMOOTHING / (size - 2)
    c1 = (size - 2) * smooth * math.log(smooth) + _CONF * math.log(_CONF)
    t32 = target.astype(jnp.int32)
    g, x0 = _sc_gather(x, t32)
    out = pl.pallas_call(
        _make_tc_body(b, size, n_steps, smooth, c1),
        grid=(n_steps,),
        in_specs=[
            pl.BlockSpec((b, 1), lambda c: (0, 0)),
            pl.BlockSpec((b, 1), lambda c: (0, 0)),
            pl.BlockSpec((b, 1), lambda c: (0, 0)),
            pl.BlockSpec((_RB, size), lambda c: (c, 0)),
        ],
        out_specs=pl.BlockSpec((1, 1), lambda c: (0, 0), memory_space=pltpu.SMEM),
        out_shape=jax.ShapeDtypeStruct((1, 1), jnp.float32),
        scratch_shapes=[pltpu.VMEM((b, 1), jnp.float32)],
        compiler_params=pltpu.CompilerParams(
            dimension_semantics=("arbitrary",),
        ),
    )(t32.reshape(b, 1), g.reshape(b, 1), x0.reshape(b, 1), x)
    return out[0, 0]
